# baseline (speedup 1.0000x reference)
import functools

import jax
import jax.numpy as jnp
from jax import lax
from jax.experimental import pallas as pl
from jax.experimental.pallas import tpu as pltpu
from jax.experimental.pallas import tpu_sc as plsc

_BATCH = 16384


@functools.partial(jax.jit, static_argnames=())
def _gather_sc(states, V):
    info = plsc.get_sparse_core_info()
    num_cores = 1
    nw = num_cores * info.num_subcores
    b_per_w = _BATCH // nw
    mesh = plsc.VectorSubcoreMesh(
        core_axis_name="c", subcore_axis_name="s", num_cores=num_cores)

    @functools.partial(
        pl.kernel,
        mesh=mesh,
        out_type=jax.ShapeDtypeStruct((_BATCH,), jnp.float32),
        scratch_types=[
            pltpu.VMEM((b_per_w,), jnp.int32),
            pltpu.VMEM((b_per_w,), jnp.float32),
            pltpu.SemaphoreType.DMA,
            pltpu.SemaphoreType.DMA,
            pltpu.SemaphoreType.DMA,
        ],
    )
    def body(states_hbm, table_hbm, out_hbm, idx_v, vals_v, *sems):
        wid = lax.axis_index("s") * num_cores + lax.axis_index("c")
        base = wid * b_per_w
        bounds = [0, (3 * b_per_w) // 8, (6 * b_per_w) // 8, b_per_w]
        spans = [(bounds[j], bounds[j + 1] - bounds[j])
                 for j in range(len(bounds) - 1)]
        stage = pltpu.async_copy(
            states_hbm.at[pl.ds(base, b_per_w)], idx_v, sems[0])
        stage.wait()
        gathers = []
        for j, (lo, ln) in enumerate(spans):
            gathers.append(pltpu.async_copy(
                table_hbm.at[idx_v.at[pl.ds(lo, ln)]],
                vals_v.at[pl.ds(lo, ln)], sems[j]))
        outs = []
        for j, (lo, ln) in enumerate(spans):
            gathers[j].wait()
            outs.append(pltpu.async_copy(
                vals_v.at[pl.ds(lo, ln)],
                out_hbm.at[pl.ds(base + lo, ln)], sems[j]))
        for o in outs:
            o.wait()

    return body(states, V)


def kernel(states, V):
    return _gather_sc(states.astype(jnp.int32), V)


# speedup vs baseline: 1.0035x; 1.0035x over previous
import functools

import jax
import jax.numpy as jnp
from jax import lax
from jax.experimental import pallas as pl
from jax.experimental.pallas import tpu as pltpu
from jax.experimental.pallas import tpu_sc as plsc

_BATCH = 16384


@functools.partial(jax.jit, static_argnames=())
def _gather_sc(states, V):
    info = plsc.get_sparse_core_info()
    num_cores = 1
    nw = num_cores * info.num_subcores
    b_per_w = _BATCH // nw
    half = b_per_w // 2
    mesh = plsc.VectorSubcoreMesh(
        core_axis_name="c", subcore_axis_name="s", num_cores=num_cores)

    @functools.partial(
        pl.kernel,
        mesh=mesh,
        out_type=jax.ShapeDtypeStruct((_BATCH,), jnp.float32),
        scratch_types=[
            pltpu.VMEM((b_per_w,), jnp.int32),
            pltpu.VMEM((b_per_w,), jnp.float32),
            pltpu.SemaphoreType.DMA,
            pltpu.SemaphoreType.DMA,
        ],
    )
    def body(states_hbm, table_hbm, out_hbm, idx_v, vals_v, sem0, sem1):
        wid = lax.axis_index("s") * num_cores + lax.axis_index("c")
        base = wid * b_per_w
        stage = pltpu.async_copy(
            states_hbm.at[pl.ds(base, b_per_w)], idx_v, sem0)
        stage.wait()
        g0 = pltpu.async_copy(
            table_hbm.at[idx_v.at[pl.ds(0, half)]],
            vals_v.at[pl.ds(0, half)], sem0)
        g1 = pltpu.async_copy(
            table_hbm.at[idx_v.at[pl.ds(half, half)]],
            vals_v.at[pl.ds(half, half)], sem1)
        g0.wait()
        g1.wait()
        out = pltpu.async_copy(
            vals_v, out_hbm.at[pl.ds(base, b_per_w)], sem0)
        out.wait()

    return body(states, V)


def kernel(states, V):
    return _gather_sc(states.astype(jnp.int32), V)


# speedup vs baseline: 1.0058x; 1.0024x over previous
import functools

import jax
import jax.numpy as jnp
from jax import lax
from jax.experimental import pallas as pl
from jax.experimental.pallas import tpu as pltpu
from jax.experimental.pallas import tpu_sc as plsc

_BATCH = 16384


@functools.partial(jax.jit, static_argnames=())
def _gather_sc(states, V):
    info = plsc.get_sparse_core_info()
    num_cores = 1
    nw = num_cores * info.num_subcores
    b_per_w = _BATCH // nw
    half = b_per_w // 2
    mesh = plsc.VectorSubcoreMesh(
        core_axis_name="c", subcore_axis_name="s", num_cores=num_cores)

    @functools.partial(
        pl.kernel,
        mesh=mesh,
        out_type=jax.ShapeDtypeStruct((_BATCH,), jnp.float32),
        scratch_types=[
            pltpu.VMEM((b_per_w,), jnp.int32),
            pltpu.VMEM((b_per_w,), jnp.float32),
            pltpu.SemaphoreType.DMA,
            pltpu.SemaphoreType.DMA,
        ],
    )
    def body(states_hbm, table_hbm, out_hbm, idx_v, vals_v, sem0, sem1):
        wid = lax.axis_index("s") * num_cores + lax.axis_index("c")
        base = wid * b_per_w
        s0 = pltpu.async_copy(
            states_hbm.at[pl.ds(base, half)], idx_v.at[pl.ds(0, half)], sem0)
        s1 = pltpu.async_copy(
            states_hbm.at[pl.ds(base + half, half)],
            idx_v.at[pl.ds(half, half)], sem1)
        s0.wait()
        g0 = pltpu.async_copy(
            table_hbm.at[idx_v.at[pl.ds(0, half)]],
            vals_v.at[pl.ds(0, half)], sem0)
        s1.wait()
        g1 = pltpu.async_copy(
            table_hbm.at[idx_v.at[pl.ds(half, half)]],
            vals_v.at[pl.ds(half, half)], sem1)
        g0.wait()
        o0 = pltpu.async_copy(
            vals_v.at[pl.ds(0, half)],
            out_hbm.at[pl.ds(base, half)], sem0)
        g1.wait()
        o1 = pltpu.async_copy(
            vals_v.at[pl.ds(half, half)],
            out_hbm.at[pl.ds(base + half, half)], sem1)
        o0.wait()
        o1.wait()

    return body(states, V)


def kernel(states, V):
    return _gather_sc(states.astype(jnp.int32), V)
